# baseline (speedup 1.0000x reference)
import functools

import jax
import jax.numpy as jnp
from jax import lax
from jax.experimental import pallas as pl
from jax.experimental.pallas import tpu as pltpu
from jax.experimental.pallas import tpu_sc as plsc

_DIM = 128
_NC = 2
_NS = 16
_NW = _NC * _NS
_CHUNK = 128
_TC_CH = 512
_N_SLICES = 2


def _sc_gather_fn(n_ent, n_rel):
    e_rows_w = n_ent // _NW
    r_rows_w = n_rel // _NW
    e_chunks = e_rows_w // _CHUNK
    r_chunks = r_rows_w // _CHUNK
    mesh = plsc.VectorSubcoreMesh(core_axis_name="c", subcore_axis_name="s")

    @functools.partial(
        pl.kernel,
        out_type=[
            jax.ShapeDtypeStruct((n_ent, _DIM), jnp.float32),
            jax.ShapeDtypeStruct((n_rel, _DIM), jnp.float32),
        ],
        mesh=mesh,
        scratch_types=[
            pltpu.VMEM((e_chunks, _CHUNK), jnp.int32),
            pltpu.VMEM((r_chunks, _CHUNK), jnp.int32),
            pltpu.VMEM((e_rows_w, _DIM), jnp.float32),
            pltpu.VMEM((r_rows_w, _DIM), jnp.float32),
            pltpu.SemaphoreType.DMA,
        ],
    )
    def gather(ent_hbm, rel_hbm, ie_hbm, ir_hbm, oe_hbm, or_hbm,
               ie_v, ir_v, erows_v, rrows_v, sem):
        wid = lax.axis_index("s") * _NC + lax.axis_index("c")
        pltpu.sync_copy(ie_hbm.at[pl.ds(wid * e_chunks, e_chunks)], ie_v)
        pltpu.sync_copy(ir_hbm.at[pl.ds(wid * r_chunks, r_chunks)], ir_v)
        copies = []
        for j in range(e_chunks):
            copies.append(pltpu.async_copy(
                ent_hbm.at[ie_v.at[j]],
                erows_v.at[pl.ds(j * _CHUNK, _CHUNK)], sem))
        for j in range(r_chunks):
            copies.append(pltpu.async_copy(
                rel_hbm.at[ir_v.at[j]],
                rrows_v.at[pl.ds(j * _CHUNK, _CHUNK)], sem))
        for c in copies:
            c.wait()
        pltpu.sync_copy(erows_v, oe_hbm.at[pl.ds(wid * e_rows_w, e_rows_w)])
        pltpu.sync_copy(rrows_v, or_hbm.at[pl.ds(wid * r_rows_w, r_rows_w)])

    return gather


def _unit(x):
    n = jnp.sqrt(jnp.sum(x * x, axis=1, keepdims=True))
    return x / jnp.maximum(n, 1e-12)


def _partial_sum(erows_ref, rrows_ref):
    hp = _unit(erows_ref[0])
    tp = _unit(erows_ref[1])
    hn = _unit(erows_ref[2])
    tn = _unit(erows_ref[3])
    rp = _unit(rrows_ref[0])
    rn = _unit(rrows_ref[1])
    dp = hp + rp - tp
    dn = hn + rn - tn
    ep = jnp.sqrt(jnp.sum(dp * dp, axis=1))
    en = jnp.sqrt(jnp.sum(dn * dn, axis=1))
    return jnp.sum(jnp.maximum(1.0 + ep - en, 0.0))


def _tc_first(erows_ref, rrows_ref, out_ref):
    i = pl.program_id(0)

    @pl.when(i == 0)
    def _():
        out_ref[...] = jnp.zeros((1, 1), jnp.float32)

    out_ref[...] += _partial_sum(erows_ref, rrows_ref).reshape(1, 1)


def _tc_last_fn(inv_b):
    def _tc_last(erows_ref, rrows_ref, prev_ref, out_ref):
        i = pl.program_id(0)

        @pl.when(i == 0)
        def _():
            out_ref[...] = prev_ref[...]

        out_ref[...] += _partial_sum(erows_ref, rrows_ref).reshape(1, 1)

        @pl.when(i == pl.num_programs(0) - 1)
        def _():
            out_ref[...] *= inv_b

    return _tc_last


def _tc_call(erows, rrows, prev, inv_b):
    bs = erows.shape[0] // 4
    erows3 = erows.reshape(4, bs, _DIM)
    rrows3 = rrows.reshape(2, bs, _DIM)
    grid = (bs // _TC_CH,)
    in_specs = [
        pl.BlockSpec((4, _TC_CH, _DIM), lambda i: (0, i, 0)),
        pl.BlockSpec((2, _TC_CH, _DIM), lambda i: (0, i, 0)),
    ]
    args = [erows3, rrows3]
    if prev is None:
        body = _tc_first
    else:
        body = _tc_last_fn(inv_b)
        in_specs.append(pl.BlockSpec((1, 1), lambda i: (0, 0)))
        args.append(prev)
    return pl.pallas_call(
        body,
        grid=grid,
        in_specs=in_specs,
        out_specs=pl.BlockSpec((1, 1), lambda i: (0, 0)),
        out_shape=jax.ShapeDtypeStruct((1, 1), jnp.float32),
    )(*args)


@jax.jit
def kernel(pos_triples, neg_triples, ent_emb, rel_emb):
    b = pos_triples.shape[0]
    bs = b // _N_SLICES
    gather = _sc_gather_fn(4 * bs, 2 * bs)

    acc = None
    for s in range(_N_SLICES):
        pos = lax.dynamic_slice_in_dim(pos_triples, s * bs, bs, axis=0)
        neg = lax.dynamic_slice_in_dim(neg_triples, s * bs, bs, axis=0)
        idx_ent = jnp.concatenate(
            [pos[:, 0], pos[:, 2], neg[:, 0], neg[:, 2]]).reshape(-1, _CHUNK)
        idx_rel = jnp.concatenate(
            [pos[:, 1], neg[:, 1]]).reshape(-1, _CHUNK)
        erows, rrows = gather(ent_emb, rel_emb, idx_ent, idx_rel)
        acc = _tc_call(erows, rrows, acc, 1.0 / b)
    return acc[0, 0]
